# baseline (speedup 1.0000x reference)
import functools
import math

import jax
import jax.numpy as jnp
from jax import lax
from jax.experimental import pallas as pl
from jax.experimental.pallas import tpu as pltpu
from jax.experimental.pallas import tpu_sc as plsc

_NBUF = 4
_CR = 8


@functools.lru_cache(maxsize=None)
def _make_gather(M, R, D, scale):
  info = plsc.get_sparse_core_info()
  NC, NS, L = info.num_cores, info.num_subcores, info.num_lanes
  NW = NC * NS
  assert M % NW == 0 and D % L == 0
  m_per = M // NW
  assert m_per % _CR == 0
  nch = m_per // _CR
  assert nch % _NBUF == 0
  ngrp = nch // _NBUF
  assert ngrp >= 2
  crows = _CR * R
  mesh = plsc.VectorSubcoreMesh(core_axis_name="c", subcore_axis_name="s")

  @functools.partial(
      pl.kernel,
      mesh=mesh,
      compiler_params=pltpu.CompilerParams(use_tc_tiling_on_sc=False),
      out_type=jax.ShapeDtypeStruct((M, R, D), jnp.float32),
      scratch_types=[
          pltpu.VMEM((m_per, R), jnp.int32),
          pltpu.VMEM((_NBUF, crows, D), jnp.float32),
      ]
      + [pltpu.SemaphoreType.DMA] * (2 * _NBUF),
  )
  def k(table_hbm, x_hbm, out_hbm, idx_v, rows_v, *sems):
    gsem = sems[:_NBUF]
    wsem = sems[_NBUF:]
    wid = lax.axis_index("s") * NC + lax.axis_index("c")
    row0 = pl.multiple_of(wid * m_per, _CR)
    pltpu.sync_copy(x_hbm.at[pl.ds(row0, m_per)], idx_v)

    def start_gather(c, j):
      for r in range(_CR):
        pltpu.async_copy(
            table_hbm.at[idx_v.at[c * _CR + r]],
            rows_v.at[j, pl.ds(r * R, R)],
            gsem[j],
        )

    def wait_gather(j):
      pltpu.make_async_copy(
          table_hbm.at[pl.ds(0, crows)], rows_v.at[j], gsem[j]
      ).wait()

    def start_write(c, j):
      for r in range(_CR):
        pltpu.async_copy(
            rows_v.at[j, pl.ds(r * R, R)],
            out_hbm.at[row0 + c * _CR + r],
            wsem[j],
        )

    def wait_write(j):
      for r in range(_CR):
        pltpu.make_async_copy(
            rows_v.at[j, pl.ds(r * R, R)], out_hbm.at[r], wsem[j]
        ).wait()

    def scale_buf(j):
      def row_body(i, carry):
        for t in range(D // L):
          sl = pl.ds(t * L, L)
          rows_v[j, i, sl] = rows_v[j, i, sl] * scale
        return carry

      lax.fori_loop(0, crows, row_body, 0, unroll=2)

    for j in range(_NBUF - 1):
      start_gather(j, j)
    for j in range(_NBUF):
      c = j
      if c + _NBUF - 1 < nch:
        if c >= 1:
          wait_write((j + _NBUF - 1) % _NBUF)
        start_gather(c + _NBUF - 1, (j + _NBUF - 1) % _NBUF)
      wait_gather(j)
      scale_buf(j)
      start_write(c, j)

    def group_body(g, carry):
      c0 = g * _NBUF
      for j in range(_NBUF):
        c = c0 + j
        wait_write((j + _NBUF - 1) % _NBUF)
        start_gather(c + _NBUF - 1, (j + _NBUF - 1) % _NBUF)
        wait_gather(j)
        scale_buf(j)
        start_write(c, j)
      return carry

    lax.fori_loop(1, ngrp - 1, group_body, 0)

    c0 = (ngrp - 1) * _NBUF
    for j in range(_NBUF):
      c = c0 + j
      if c + _NBUF - 1 < nch:
        wait_write((j + _NBUF - 1) % _NBUF)
        start_gather(c + _NBUF - 1, (j + _NBUF - 1) % _NBUF)
      wait_gather(j)
      scale_buf(j)
      start_write(c, j)
    for j in range(_NBUF):
      wait_write(j)

  return k


def kernel(x, table):
  M, R = x.shape
  D = table.shape[1]
  return _make_gather(M, R, D, float(math.sqrt(D)))(
      table, x.astype(jnp.int32)
  )


# speedup vs baseline: 1.0566x; 1.0566x over previous
import functools
import math

import jax
import jax.numpy as jnp
from jax import lax
from jax.experimental import pallas as pl
from jax.experimental.pallas import tpu as pltpu
from jax.experimental.pallas import tpu_sc as plsc

_NBUF = 4
_CR = 8


@functools.lru_cache(maxsize=None)
def _make_gather(M, R, D, scale):
  info = plsc.get_sparse_core_info()
  NC, NS, L = info.num_cores, info.num_subcores, info.num_lanes
  NW = NC * NS
  assert M % NW == 0 and D % L == 0
  m_per = M // NW
  assert m_per % _CR == 0
  nch = m_per // _CR
  assert nch % _NBUF == 0
  ngrp = nch // _NBUF
  assert ngrp >= 2
  crows = _CR * R
  mesh = plsc.VectorSubcoreMesh(core_axis_name="c", subcore_axis_name="s")

  @functools.partial(
      pl.kernel,
      mesh=mesh,
      compiler_params=pltpu.CompilerParams(use_tc_tiling_on_sc=False),
      out_type=jax.ShapeDtypeStruct((R, M, D), jnp.float32),
      scratch_types=[
          pltpu.VMEM((R, m_per), jnp.int32),
          pltpu.VMEM((_NBUF, crows, D), jnp.float32),
      ]
      + [pltpu.SemaphoreType.DMA] * (2 * _NBUF),
  )
  def k(table_hbm, xt_hbm, out_hbm, idx_v, rows_v, *sems):
    gsem = sems[:_NBUF]
    wsem = sems[_NBUF:]
    wid = lax.axis_index("s") * NC + lax.axis_index("c")
    col0 = pl.multiple_of(wid * m_per, _CR)
    pltpu.sync_copy(xt_hbm.at[:, pl.ds(col0, m_per)], idx_v)

    def start_gather(c, b):
      def gath(j, carry):
        pltpu.async_copy(
            table_hbm.at[idx_v.at[j, pl.ds(c * _CR, _CR)]],
            rows_v.at[b, pl.ds(j * _CR, _CR)],
            gsem[b],
        )
        return carry

      lax.fori_loop(0, R, gath, 0)

    def wait_gather(b):
      pltpu.make_async_copy(
          table_hbm.at[pl.ds(0, crows)], rows_v.at[b], gsem[b]
      ).wait()

    def start_write(c, b):
      def wr(j, carry):
        pltpu.async_copy(
            rows_v.at[b, pl.ds(j * _CR, _CR)],
            out_hbm.at[j, pl.ds(col0 + c * _CR, _CR)],
            wsem[b],
        )
        return carry

      lax.fori_loop(0, R, wr, 0)

    def wait_write(b):
      pltpu.make_async_copy(
          rows_v.at[b], out_hbm.at[0, pl.ds(0, crows)], wsem[b]
      ).wait()

    def scale_buf(b):
      def row_body(i, carry):
        for t in range(D // L):
          sl = pl.ds(t * L, L)
          rows_v[b, i, sl] = rows_v[b, i, sl] * scale
        return carry

      lax.fori_loop(0, crows, row_body, 0, unroll=2)

    for b in range(_NBUF - 1):
      start_gather(b, b)
    for b in range(_NBUF):
      c = b
      if c + _NBUF - 1 < nch:
        if c >= 1:
          wait_write((b + _NBUF - 1) % _NBUF)
        start_gather(c + _NBUF - 1, (b + _NBUF - 1) % _NBUF)
      wait_gather(b)
      scale_buf(b)
      start_write(c, b)

    def group_body(g, carry):
      c0 = g * _NBUF
      for b in range(_NBUF):
        c = c0 + b
        wait_write((b + _NBUF - 1) % _NBUF)
        start_gather(c + _NBUF - 1, (b + _NBUF - 1) % _NBUF)
        wait_gather(b)
        scale_buf(b)
        start_write(c, b)
      return carry

    lax.fori_loop(1, ngrp - 1, group_body, 0)

    c0 = (ngrp - 1) * _NBUF
    for b in range(_NBUF):
      c = c0 + b
      if c + _NBUF - 1 < nch:
        wait_write((b + _NBUF - 1) % _NBUF)
        start_gather(c + _NBUF - 1, (b + _NBUF - 1) % _NBUF)
      wait_gather(b)
      scale_buf(b)
      start_write(c, b)
    for b in range(_NBUF):
      wait_write(b)

  return k


def kernel(x, table):
  M, R = x.shape
  D = table.shape[1]
  xt = x.astype(jnp.int32).T
  o = _make_gather(M, R, D, float(math.sqrt(D)))(table, xt)
  return o.transpose((1, 0, 2))
